# Initial kernel scaffold; baseline (speedup 1.0000x reference)
import jax
import jax.numpy as jnp
from jax.experimental import pallas as pl


def kernel(instance_embedding, relation_embedding, proto_table, rel_label_ids, labels):
    raise NotImplementedError("write your pallas kernel here")


# speedup vs baseline: 10.9395x; 10.9395x over previous
import functools

import jax
import jax.numpy as jnp
from jax import lax
from jax.experimental import pallas as pl
from jax.experimental.pallas import tpu as pltpu

P = 128
H = 128
NR = 4
U = 16
N_INST = 4096
BLK = 512
GRID = N_INST // BLK
LN2 = 0.6931471805599453


def _dg(a, b, ca, cb):
    return lax.dot_general(
        a, b, (((ca,), (cb,)), ((), ())),
        precision=lax.Precision.HIGHEST,
        preferred_element_type=jnp.float32,
    )


def _tc_body(x_ref, rel_ref, proto_ref, orig_ref, lbl_ref,
             logits_ref, loss_ref, proto_out_ref, pn_ref):
    i = pl.program_id(0)

    @pl.when(i == 0)
    def _prologue():
        f32 = jnp.float32

        def fiota(shape, dim):
            return lax.broadcasted_iota(jnp.int32, shape, dim).astype(f32)

        eye = (lax.broadcasted_iota(jnp.int32, (U, U), 0) ==
               lax.broadcasted_iota(jnp.int32, (U, U), 1)).astype(f32)

        lbl_row = lbl_ref[...].astype(f32)
        lbl_col = _dg(eye, lbl_row, 1, 1)
        eq = (lbl_col == lbl_row).astype(f32)
        lower = (lax.broadcasted_iota(jnp.int32, (U, U), 1) <
                 lax.broadcasted_iota(jnp.int32, (U, U), 0)).astype(f32)
        first_col = (jnp.sum(eq * lower, axis=1, keepdims=True) == 0.0).astype(f32)
        first_row = _dg(first_col, eye, 0, 0)
        less = (lbl_row < lbl_col).astype(f32)
        rank_col = jnp.sum(first_row * less, axis=1, keepdims=True)
        rank_row = _dg(rank_col, eye, 0, 0)
        n_u = jnp.sum(first_col)
        iota_col = fiota((U, 1), 0)
        sel = (rank_row == fiota((U, U), 0)).astype(f32) * first_row
        uniq_col = jnp.sum(sel * lbl_row, axis=1, keepdims=True)
        uniq_col = jnp.where(iota_col >= n_u, float(P), uniq_col)
        uniq_row = _dg(uniq_col, eye, 0, 0)
        valid_col = (uniq_col < float(P)).astype(f32)
        n_valid = jnp.sum(valid_col)

        o = orig_ref[...].astype(f32)
        q_iota = fiota((NR * U, P), 1)
        dacc = jnp.zeros((NR * U, P), f32)
        for p in range(P):
            dacc = dacc + (o[:, p:p + 1] == q_iota).astype(f32)
        s64 = jnp.sum(o, axis=1, keepdims=True)
        kv = s64[0:U] + s64[U:2 * U] + s64[2 * U:3 * U] + s64[3 * U:4 * U]
        az = (jnp.sum(kv) == 0.0).astype(f32)
        k64 = jnp.concatenate([kv, kv, kv, kv], axis=0)
        dg64 = dacc * (s64 != 0.0).astype(f32) * (k64 != 0.0).astype(f32)

        e_mat = (rank_col == fiota((U, U), 1)).astype(f32)

        gc = (fiota((P, U), 0) ==
              jnp.minimum(uniq_row, float(P - 1))).astype(f32)
        proto = proto_ref[...]
        t_emb = _dg(gc, proto, 0, 0)

        loss_sum = 0.0
        n_eff = 0.0
        prop = jnp.zeros((U, H), f32)
        num_prop = jnp.zeros((U, 1), f32)
        for r in range(NR):
            proj_r = _dg(proto, rel_ref[r], 1, 0)
            d_r = dg64[r * U:(r + 1) * U]
            c_r = _dg(e_mat, d_r, 1, 0) * valid_col
            z_r = _dg(t_emb, proj_r, 1, 1)
            sim_r = 1.0 / (1.0 + jnp.exp(-z_r))
            loss_sum = loss_sum + jnp.sum(
                c_r * jnp.log(1.0 + jnp.exp(1.0 - 2.0 * sim_r)))
            n_eff = n_eff + jnp.sum(c_r)
            prop = prop + _dg(c_r, proj_r, 1, 0)
            num_prop = num_prop + jnp.sum(c_r, axis=1, keepdims=True)

        denom = jnp.where(num_prop > 0.0, num_prop, 1.0)
        upd = 0.5 * t_emb + 0.5 * prop / denom
        new_rows = jnp.where(num_prop > 0.0, upd, t_emb)
        delta = (new_rows - t_emb) * valid_col * (1.0 - az)
        sc_t = (fiota((P, U), 0) == uniq_row).astype(f32)
        proto_out = proto + _dg(sc_t, delta, 1, 0)
        proto_out_ref[...] = proto_out

        tot = n_valid * float(NR * P)
        loss = (1.0 - az) * ((loss_sum + (tot - n_eff) * LN2) / tot)
        loss_ref[...] = jnp.broadcast_to(loss, (1, 1))

        one_row = jnp.ones((1, P), f32)
        pn_ref[...] = _dg(one_row, proto_out * proto_out, 1, 1)

    x = x_ref[...]
    xn = jnp.sum(x * x, axis=1, keepdims=True)
    cross = _dg(x, proto_out_ref[...], 1, 1)
    logits_ref[...] = 2.0 * cross - xn - pn_ref[...]


@functools.partial(jax.jit, static_argnames=('interpret',))
def _run(x, rel, proto, orig64, lbl2d, interpret=False):
    return pl.pallas_call(
        _tc_body,
        grid=(GRID,),
        in_specs=[
            pl.BlockSpec((BLK, H), lambda i: (i, 0)),
            pl.BlockSpec((NR, H, H), lambda i: (0, 0, 0)),
            pl.BlockSpec((P, H), lambda i: (0, 0)),
            pl.BlockSpec((NR * U, P), lambda i: (0, 0)),
            pl.BlockSpec((1, U), lambda i: (0, 0)),
        ],
        out_specs=[
            pl.BlockSpec((BLK, P), lambda i: (i, 0)),
            pl.BlockSpec((1, 1), lambda i: (0, 0)),
            pl.BlockSpec((P, H), lambda i: (0, 0)),
        ],
        out_shape=[
            jax.ShapeDtypeStruct((N_INST, P), jnp.float32),
            jax.ShapeDtypeStruct((1, 1), jnp.float32),
            jax.ShapeDtypeStruct((P, H), jnp.float32),
        ],
        scratch_shapes=[pltpu.VMEM((1, P), jnp.float32)],
        interpret=interpret,
    )(x, rel, proto, orig64, lbl2d)


def kernel(instance_embedding, relation_embedding, proto_table, rel_label_ids, labels):
    orig64 = rel_label_ids.astype(jnp.int32).transpose(1, 0, 2).reshape(NR * U, P)
    lbl2d = labels.astype(jnp.int32).reshape(1, U)
    logits, loss, proto_out = _run(
        instance_embedding, relation_embedding, proto_table, orig64, lbl2d)
    return (logits, loss.reshape(()), proto_out)
